# baseline (speedup 1.0000x reference)
import jax
import jax.numpy as jnp
from jax.experimental import pallas as pl
from jax.experimental.pallas import tpu as pltpu

_N = 1_000_000
_D = 64
_NPACK = _N // 2
_BLOCK_ROWS = 4_000


def _rowdot_body(a_ref, b_ref, o_ref):
    p = a_ref[0] * b_ref[0]
    lane = jax.lax.broadcasted_iota(jnp.int32, (128, 2), 0)
    col = jax.lax.broadcasted_iota(jnp.int32, (128, 2), 1)
    sel = ((lane // _D) == col).astype(jnp.float32)
    o_ref[0] = jax.lax.dot_general(
        p, sel, (((1,), (0,)), ((), ())), preferred_element_type=jnp.float32
    )


def kernel(gut, git):
    n_blocks = _NPACK // _BLOCK_ROWS
    a3 = gut.reshape(n_blocks, _BLOCK_ROWS, 2 * _D)
    b3 = git.reshape(n_blocks, _BLOCK_ROWS, 2 * _D)
    out = pl.pallas_call(
        _rowdot_body,
        grid=(n_blocks,),
        in_specs=[
            pl.BlockSpec((1, _BLOCK_ROWS, 2 * _D), lambda i: (i, 0, 0)),
            pl.BlockSpec((1, _BLOCK_ROWS, 2 * _D), lambda i: (i, 0, 0)),
        ],
        out_specs=pl.BlockSpec((1, _BLOCK_ROWS, 2), lambda i: (i, 0, 0)),
        out_shape=jax.ShapeDtypeStruct((n_blocks, _BLOCK_ROWS, 2), jnp.float32),
        compiler_params=pltpu.CompilerParams(
            dimension_semantics=("arbitrary",),
        ),
    )(a3, b3)
    return out.reshape(_N)


# speedup vs baseline: 1.1783x; 1.1783x over previous
import jax
import jax.numpy as jnp
from jax.experimental import pallas as pl
from jax.experimental.pallas import tpu as pltpu

_N = 1_000_000
_D = 64
_BLOCK_ROWS = 8_000


def _rowdot_body(a_ref, b_ref, o_ref):
    p = a_ref[...] * b_ref[...]
    o_ref[...] = jnp.sum(p, axis=-1, keepdims=True)


def kernel(gut, git):
    n_blocks = _N // _BLOCK_ROWS
    out = pl.pallas_call(
        _rowdot_body,
        grid=(n_blocks,),
        in_specs=[
            pl.BlockSpec((_BLOCK_ROWS, _D), lambda i: (i, 0)),
            pl.BlockSpec((_BLOCK_ROWS, _D), lambda i: (i, 0)),
        ],
        out_specs=pl.BlockSpec((_BLOCK_ROWS, 1), lambda i: (i, 0)),
        out_shape=jax.ShapeDtypeStruct((_N, 1), jnp.float32),
        compiler_params=pltpu.CompilerParams(
            dimension_semantics=("arbitrary",),
        ),
    )(gut, git)
    return out.reshape(_N)


# speedup vs baseline: 1.5567x; 1.3211x over previous
import jax
import jax.numpy as jnp
from jax.experimental import pallas as pl
from jax.experimental.pallas import tpu as pltpu

_N = 1_000_000
_D = 64
_BLOCK_ROWS = 16_384


def _rowdot_body(a_ref, b_ref, o_ref):
    p = a_ref[...] * b_ref[...]
    o_ref[...] = jnp.sum(p.T, axis=0)


def kernel(gut, git):
    n_blocks = pl.cdiv(_N, _BLOCK_ROWS)
    out = pl.pallas_call(
        _rowdot_body,
        grid=(n_blocks,),
        in_specs=[
            pl.BlockSpec((_BLOCK_ROWS, _D), lambda i: (i, 0)),
            pl.BlockSpec((_BLOCK_ROWS, _D), lambda i: (i, 0)),
        ],
        out_specs=pl.BlockSpec((_BLOCK_ROWS,), lambda i: (i,)),
        out_shape=jax.ShapeDtypeStruct((_N,), jnp.float32),
        compiler_params=pltpu.CompilerParams(
            dimension_semantics=("arbitrary",),
        ),
    )(gut, git)
    return out
